# baseline (speedup 1.0000x reference)
import functools

import jax
import jax.numpy as jnp
from jax import lax
from jax.experimental import pallas as pl
from jax.experimental.pallas import tpu as pltpu
from jax.experimental.pallas import tpu_sc as plsc

BATCH = 16384
FEAT = 640
NCLASS = 6
CPAD = 8


NC = 2
NS = 16
NW = NC * NS
RPW = BATCH // NW
CH = 64
NCH = RPW // CH
NVR = FEAT // 16
NACC = 4

_sc_mesh = plsc.VectorSubcoreMesh(core_axis_name="c", subcore_axis_name="s")


@functools.partial(
    pl.kernel,
    out_type=jax.ShapeDtypeStruct((NW, 32), jnp.float32),
    mesh=_sc_mesh,
    scratch_types=[
        pltpu.VMEM((CH * FEAT,), jnp.float32),
        pltpu.VMEM((CH * FEAT,), jnp.float32),
        pltpu.VMEM((NCLASS * FEAT,), jnp.float32),
        pltpu.VMEM((RPW,), jnp.int32),
        pltpu.VMEM((16,), jnp.float32),
        pltpu.VMEM((32,), jnp.float32),
        pltpu.SemaphoreType.DMA,
        pltpu.SemaphoreType.DMA,
    ],
)
def _sc_loss(f_hbm, lab_hbm, c_hbm, out_hbm,
             fbuf0, fbuf1, cbuf, labbuf, accbuf, obuf, sem0, sem1):
    wid = lax.axis_index("s") * NC + lax.axis_index("c")
    rbase = wid * RPW

    pltpu.sync_copy(c_hbm, cbuf)
    pltpu.sync_copy(lab_hbm.at[pl.ds(rbase, RPW)], labbuf)

    bufs = (fbuf0, fbuf1)
    sems = (sem0, sem1)
    pltpu.async_copy(f_hbm.at[pl.ds(rbase * FEAT, CH * FEAT)], fbuf0, sem0)
    pltpu.async_copy(f_hbm.at[pl.ds((rbase + CH) * FEAT, CH * FEAT)],
                     fbuf1, sem1)

    def _cnt_body(i, cnt):
        labv = labbuf[pl.ds(i * 16, 16)]
        return cnt + jnp.where(labv < NCLASS,
                               jnp.float32(1.0), jnp.float32(0.0))
    cnt = lax.fori_loop(0, RPW // 16, _cnt_body,
                        jnp.zeros((16,), jnp.float32))

    accbuf[...] = jnp.zeros((16,), jnp.float32)

    def _compute_chunk(buf, ch):
        def _group_body(g, carry):
            labv = labbuf[pl.ds(ch * CH + g * 16, 16)]
            maskv = jnp.where(labv < NCLASS,
                              jnp.float32(1.0), jnp.float32(0.0))
            coffv = jnp.minimum(labv, NCLASS - 1) * FEAT
            for t in range(16):
                coff = coffv[t]
                m = maskv[t]
                foff = (g * 16 + t) * FEAT
                ra = [jnp.zeros((16,), jnp.float32) for _ in range(NACC)]
                for j in range(NVR):
                    d = (buf[pl.ds(foff + j * 16, 16)]
                         - cbuf[pl.ds(coff + j * 16, 16)])
                    ra[j % NACC] = ra[j % NACC] + d * d
                rowv = ((ra[0] + ra[1]) + (ra[2] + ra[3])) * m
                plsc.addupdate(accbuf.at[pl.ds(0, 16)], rowv)
            return carry

        lax.fori_loop(0, CH // 16, _group_body, 0)

    def _pair_body(p, carry):
        for b in range(2):
            ch = 2 * p + b
            pltpu.make_async_copy(
                f_hbm.at[pl.ds(0, CH * FEAT)], bufs[b], sems[b]).wait()
            _compute_chunk(bufs[b], ch)

            @pl.when(ch + 2 < NCH)
            def _():
                pltpu.async_copy(
                    f_hbm.at[pl.ds((rbase + (ch + 2) * CH) * FEAT,
                                   CH * FEAT)],
                    bufs[b], sems[b])
        return carry

    lax.fori_loop(0, NCH // 2, _pair_body, 0)

    obuf[pl.ds(0, 16)] = accbuf[...]
    obuf[pl.ds(16, 16)] = cnt
    pltpu.sync_copy(obuf, out_hbm.at[wid])


@jax.jit
def _center_loss_sc(features, labels, centers):
    part = _sc_loss(features.reshape(-1), labels, centers.reshape(-1))
    return jnp.sum(part[:, :16]) / jnp.sum(part[:, 16:])


BB = 2048
NB = BATCH // BB


def _tc_body(f_ref, lab_ref, ct_ref, out_ref, acc_ref):
    i = pl.program_id(0)

    @pl.when(i == 0)
    def _():
        acc_ref[0] = 0.0
        acc_ref[1] = 0.0

    f = f_ref[...]
    lab = lab_ref[...]
    ct = ct_ref[...]

    mask = (lab < NCLASS).astype(jnp.float32)
    onehot = (lab == lax.broadcasted_iota(jnp.int32, (BB, CPAD), 1))
    onehot = onehot.astype(jnp.float32) * mask

    p = jnp.dot(f, ct, preferred_element_type=jnp.float32)
    c2 = jnp.sum(ct * ct, axis=0, keepdims=True)
    rows2 = jnp.sum(f * f, axis=1, keepdims=True)

    contrib = jnp.sum(rows2 * mask) + jnp.sum(onehot * (c2 - 2.0 * p))
    acc_ref[0] += contrib
    acc_ref[1] += jnp.sum(mask)

    @pl.when(i == NB - 1)
    def _():
        out_ref[0, 0] = acc_ref[0] / acc_ref[1]


@jax.jit
def _center_loss_tc(features, labels, centers_t):
    lab2d = labels.reshape(BATCH, 1)
    out = pl.pallas_call(
        _tc_body,
        grid=(NB,),
        in_specs=[
            pl.BlockSpec((BB, FEAT), lambda i: (i, 0)),
            pl.BlockSpec((BB, 1), lambda i: (i, 0)),
            pl.BlockSpec((FEAT, CPAD), lambda i: (0, 0)),
        ],
        out_specs=pl.BlockSpec(memory_space=pltpu.SMEM),
        out_shape=jax.ShapeDtypeStruct((1, 1), jnp.float32),
        scratch_shapes=[pltpu.SMEM((2,), jnp.float32)],
    )(features, lab2d, centers_t)
    return out[0, 0]


def kernel(features, labels, centers):
    return _center_loss_sc(features, labels, centers)
